# baseline (speedup 1.0000x reference)
import dataclasses

import jax
import jax.numpy as jnp
from jax import lax
from jax.experimental import pallas as pl
from jax.experimental.pallas import tpu as pltpu
from jax.experimental.pallas import tpu_sc as plsc

_cp = pltpu.CompilerParams()
if "needs_layout_passes" in pltpu.CompilerParams.__dataclass_fields__:
    _cp = dataclasses.replace(_cp, needs_layout_passes=False)

_N = 100000
_D = 128
_V = 109
_BIG = 3128
_SMALL = 3120
_C = 400
_S = 256
_NMAIN = 7

_mesh = plsc.VectorSubcoreMesh(core_axis_name="core", subcore_axis_name="subcore")


def kernel(x, table):
    idx = x.astype(jnp.int32)

    @pl.kernel(
        out_type=jax.ShapeDtypeStruct((_N, _D), table.dtype),
        mesh=_mesh,
        compiler_params=_cp,
        scratch_types=[
            pltpu.VMEM_SHARED((_V, _D), jnp.float32),
            pltpu.VMEM((_V, _D), jnp.float32),
            pltpu.VMEM((_BIG,), jnp.int32),
            pltpu.VMEM((_C, _D), jnp.float32),
            pltpu.VMEM((_C, _D), jnp.float32),
            pltpu.SemaphoreType.DMA,
            pltpu.SemaphoreType.DMA,
            pltpu.SemaphoreType.DMA,
            pltpu.SemaphoreType.DMA,
            pltpu.SemaphoreType.DMA,
        ],
    )
    def _gather(table_hbm, i_hbm, o_hbm, table_sh, table_v, idx_v,
                buf0, buf1, g0, g1, w0, w1, tsem):
        w = lax.axis_index("subcore") * 2 + lax.axis_index("core")
        base = pl.multiple_of(w * _SMALL + 8 * jnp.minimum(w, 20), 8)

        @pl.when(lax.axis_index("subcore") == 0)
        def _():
            pltpu.async_copy(table_hbm, table_sh, tsem).wait()

        @pl.when(w < 20)
        def _():
            pltpu.sync_copy(i_hbm.at[pl.ds(base, _BIG)], idx_v)

        @pl.when(w >= 20)
        def _():
            pltpu.sync_copy(i_hbm.at[pl.ds(base, _SMALL)],
                            idx_v.at[pl.ds(0, _SMALL)])

        plsc.subcore_barrier()
        pltpu.sync_copy(table_sh, table_v)

        bufs = (buf0, buf1)
        gsems = (g0, g1)
        wsems = (w0, w1)

        def start_gather(k, buf, gsem):
            pltpu.async_copy(
                table_sh.at[idx_v.at[pl.ds(k * _C, _S)]],
                buf.at[pl.ds(0, _S)], gsem)

        def wait_gather(buf, gsem):
            pltpu.make_async_copy(table_sh.at[idx_v.at[pl.ds(0, _S)]],
                                  buf.at[pl.ds(0, _S)], gsem).wait()

        lane = lax.iota(jnp.int32, 16)

        def vpu_fill(k, buf):
            @pl.loop(0, (_C - _S) // 16)
            def _(i16):
                rvec = idx_v[pl.ds(k * _C + _S + i16 * 16, 16)]
                rowvec = lane + (_S + i16 * 16)
                for c in range(_D):
                    col = jnp.full((16,), c, jnp.int32)
                    vals = plsc.load_gather(table_v, [rvec, col])
                    plsc.store_scatter(buf, [rowvec, col], vals)

        def start_write(k, buf, wsem):
            obase = pl.multiple_of(base + k * _C, 8)
            pltpu.async_copy(buf, o_hbm.at[pl.ds(obase, _C)], wsem)

        def wait_write(buf, wsem):
            pltpu.make_async_copy(bufs[0], o_hbm.at[pl.ds(0, _C)],
                                  wsem).wait()

        start_gather(0, bufs[0], gsems[0])
        vpu_fill(0, bufs[0])
        for k in range(_NMAIN):
            j, jn = k % 2, (k + 1) % 2
            wait_gather(bufs[j], gsems[j])
            start_write(k, bufs[j], wsems[j])
            if k + 1 < _NMAIN:
                if k + 1 >= 2:
                    wait_write(bufs[jn], wsems[jn])
                start_gather(k + 1, bufs[jn], gsems[jn])
                vpu_fill(k + 1, bufs[jn])

        jt = _NMAIN % 2
        wait_write(bufs[jt], wsems[jt])
        tbase = pl.multiple_of(base + _NMAIN * _C, 8)

        @pl.when(w < 20)
        def _():
            pltpu.async_copy(
                table_sh.at[idx_v.at[pl.ds(_NMAIN * _C, _BIG - _NMAIN * _C)]],
                bufs[jt].at[pl.ds(0, _BIG - _NMAIN * _C)], gsems[jt])
            pltpu.make_async_copy(
                table_sh.at[idx_v.at[pl.ds(0, _BIG - _NMAIN * _C)]],
                bufs[jt].at[pl.ds(0, _BIG - _NMAIN * _C)], gsems[jt]).wait()
            pltpu.async_copy(bufs[jt].at[pl.ds(0, _BIG - _NMAIN * _C)],
                             o_hbm.at[pl.ds(tbase, _BIG - _NMAIN * _C)],
                             wsems[jt])

        @pl.when(w >= 20)
        def _():
            pltpu.async_copy(
                table_sh.at[idx_v.at[pl.ds(_NMAIN * _C, _SMALL - _NMAIN * _C)]],
                bufs[jt].at[pl.ds(0, _SMALL - _NMAIN * _C)], gsems[jt])
            pltpu.make_async_copy(
                table_sh.at[idx_v.at[pl.ds(0, _SMALL - _NMAIN * _C)]],
                bufs[jt].at[pl.ds(0, _SMALL - _NMAIN * _C)], gsems[jt]).wait()
            pltpu.async_copy(bufs[jt].at[pl.ds(0, _SMALL - _NMAIN * _C)],
                             o_hbm.at[pl.ds(tbase, _SMALL - _NMAIN * _C)],
                             wsems[jt])

        pltpu.make_async_copy(bufs[(_NMAIN - 1) % 2], o_hbm.at[pl.ds(0, _C)],
                              wsems[(_NMAIN - 1) % 2]).wait()

        @pl.when(w < 20)
        def _():
            pltpu.make_async_copy(bufs[jt].at[pl.ds(0, _BIG - _NMAIN * _C)],
                                  o_hbm.at[pl.ds(0, _BIG - _NMAIN * _C)],
                                  wsems[jt]).wait()

        @pl.when(w >= 20)
        def _():
            pltpu.make_async_copy(bufs[jt].at[pl.ds(0, _SMALL - _NMAIN * _C)],
                                  o_hbm.at[pl.ds(0, _SMALL - _NMAIN * _C)],
                                  wsems[jt]).wait()

    return _gather(table, idx)


# speedup vs baseline: 4.2247x; 4.2247x over previous
import jax
import jax.numpy as jnp
from jax import lax
from jax.experimental import pallas as pl
from jax.experimental.pallas import tpu as pltpu
from jax.experimental.pallas import tpu_sc as plsc

_N = 100000
_D = 128
_V = 109
_BIG = 3128
_SMALL = 3120
_C = 496
_NMAIN = 6

_mesh = plsc.VectorSubcoreMesh(core_axis_name="core", subcore_axis_name="subcore")


def kernel(x, table):
    idx = x.astype(jnp.int32)

    @pl.kernel(
        out_type=jax.ShapeDtypeStruct((_N, _D), table.dtype),
        mesh=_mesh,
        scratch_types=[
            pltpu.VMEM_SHARED((_V, _D), jnp.float32),
            pltpu.VMEM((_BIG,), jnp.int32),
            pltpu.VMEM((_C, _D), jnp.float32),
            pltpu.VMEM((_C, _D), jnp.float32),
            pltpu.SemaphoreType.DMA,
            pltpu.SemaphoreType.DMA,
            pltpu.SemaphoreType.DMA,
            pltpu.SemaphoreType.DMA,
            pltpu.SemaphoreType.DMA,
        ],
    )
    def _gather(table_hbm, i_hbm, o_hbm, table_sh, idx_v, buf0, buf1,
                g0, g1, w0, w1, tsem):
        w = lax.axis_index("subcore") * 2 + lax.axis_index("core")
        base = pl.multiple_of(w * _SMALL + 8 * jnp.minimum(w, 20), 8)

        @pl.when(lax.axis_index("subcore") == 0)
        def _():
            pltpu.async_copy(table_hbm, table_sh, tsem).wait()

        @pl.when(w < 20)
        def _():
            pltpu.sync_copy(i_hbm.at[pl.ds(base, _BIG)], idx_v)

        @pl.when(w >= 20)
        def _():
            pltpu.sync_copy(i_hbm.at[pl.ds(base, _SMALL)],
                            idx_v.at[pl.ds(0, _SMALL)])

        plsc.subcore_barrier()

        bufs = (buf0, buf1)
        gsems = (g0, g1)
        wsems = (w0, w1)

        def start_gather(k, buf, gsem):
            pltpu.async_copy(
                table_sh.at[idx_v.at[pl.ds(k * _C, _C)]], buf, gsem)

        def start_write(k, buf, wsem):
            obase = pl.multiple_of(base + k * _C, 8)
            pltpu.async_copy(buf, o_hbm.at[pl.ds(obase, _C)], wsem)

        start_gather(0, bufs[0], gsems[0])
        for k in range(_NMAIN):
            j, jn = k % 2, (k + 1) % 2
            pltpu.make_async_copy(table_sh.at[idx_v.at[pl.ds(0, _C)]],
                                  bufs[j], gsems[j]).wait()
            start_write(k, bufs[j], wsems[j])
            if k + 1 < _NMAIN:
                if k + 1 >= 2:
                    pltpu.make_async_copy(bufs[jn],
                                          o_hbm.at[pl.ds(0, _C)],
                                          wsems[jn]).wait()
                start_gather(k + 1, bufs[jn], gsems[jn])

        jt = _NMAIN % 2
        pltpu.make_async_copy(bufs[jt], o_hbm.at[pl.ds(0, _C)],
                              wsems[jt]).wait()
        tbase = pl.multiple_of(base + _NMAIN * _C, 8)

        @pl.when(w < 20)
        def _():
            pltpu.async_copy(
                table_sh.at[idx_v.at[pl.ds(_NMAIN * _C, _BIG - _NMAIN * _C)]],
                bufs[jt].at[pl.ds(0, _BIG - _NMAIN * _C)], gsems[jt])
            pltpu.make_async_copy(
                table_sh.at[idx_v.at[pl.ds(0, _BIG - _NMAIN * _C)]],
                bufs[jt].at[pl.ds(0, _BIG - _NMAIN * _C)], gsems[jt]).wait()
            pltpu.async_copy(bufs[jt].at[pl.ds(0, _BIG - _NMAIN * _C)],
                             o_hbm.at[pl.ds(tbase, _BIG - _NMAIN * _C)],
                             wsems[jt])

        @pl.when(w >= 20)
        def _():
            pltpu.async_copy(
                table_sh.at[idx_v.at[pl.ds(_NMAIN * _C, _SMALL - _NMAIN * _C)]],
                bufs[jt].at[pl.ds(0, _SMALL - _NMAIN * _C)], gsems[jt])
            pltpu.make_async_copy(
                table_sh.at[idx_v.at[pl.ds(0, _SMALL - _NMAIN * _C)]],
                bufs[jt].at[pl.ds(0, _SMALL - _NMAIN * _C)], gsems[jt]).wait()
            pltpu.async_copy(bufs[jt].at[pl.ds(0, _SMALL - _NMAIN * _C)],
                             o_hbm.at[pl.ds(tbase, _SMALL - _NMAIN * _C)],
                             wsems[jt])

        pltpu.make_async_copy(bufs[(_NMAIN - 1) % 2], o_hbm.at[pl.ds(0, _C)],
                              wsems[(_NMAIN - 1) % 2]).wait()

        @pl.when(w < 20)
        def _():
            pltpu.make_async_copy(bufs[jt].at[pl.ds(0, _BIG - _NMAIN * _C)],
                                  o_hbm.at[pl.ds(0, _BIG - _NMAIN * _C)],
                                  wsems[jt]).wait()

        @pl.when(w >= 20)
        def _():
            pltpu.make_async_copy(bufs[jt].at[pl.ds(0, _SMALL - _NMAIN * _C)],
                                  o_hbm.at[pl.ds(0, _SMALL - _NMAIN * _C)],
                                  wsems[jt]).wait()

    return _gather(table, idx)


# speedup vs baseline: 4.2444x; 1.0047x over previous
import jax
import jax.numpy as jnp
from jax import lax
from jax.experimental import pallas as pl
from jax.experimental.pallas import tpu as pltpu
from jax.experimental.pallas import tpu_sc as plsc

_N = 100000
_D = 128
_V = 109
_BIG = 3128
_SMALL = 3120
_C = 400
_NMAIN = 7

_mesh = plsc.VectorSubcoreMesh(core_axis_name="core", subcore_axis_name="subcore")


def kernel(x, table):
    idx = x.astype(jnp.int32)

    @pl.kernel(
        out_type=jax.ShapeDtypeStruct((_N, _D), table.dtype),
        mesh=_mesh,
        scratch_types=[
            pltpu.VMEM_SHARED((_V, _D), jnp.float32),
            pltpu.VMEM((_BIG,), jnp.int32),
            pltpu.VMEM((_C, _D), jnp.float32),
            pltpu.VMEM((_C, _D), jnp.float32),
            pltpu.SemaphoreType.DMA,
            pltpu.SemaphoreType.DMA,
            pltpu.SemaphoreType.DMA,
            pltpu.SemaphoreType.DMA,
            pltpu.SemaphoreType.DMA,
        ],
    )
    def _gather(table_hbm, i_hbm, o_hbm, table_sh, idx_v, buf0, buf1,
                g0, g1, w0, w1, tsem):
        w = lax.axis_index("subcore") * 2 + lax.axis_index("core")
        base = pl.multiple_of(w * _SMALL + 8 * jnp.minimum(w, 20), 8)

        @pl.when(lax.axis_index("subcore") == 0)
        def _():
            pltpu.async_copy(table_hbm, table_sh, tsem).wait()

        @pl.when(w < 20)
        def _():
            pltpu.sync_copy(i_hbm.at[pl.ds(base, _BIG)], idx_v)

        @pl.when(w >= 20)
        def _():
            pltpu.sync_copy(i_hbm.at[pl.ds(base, _SMALL)],
                            idx_v.at[pl.ds(0, _SMALL)])

        plsc.subcore_barrier()

        bufs = (buf0, buf1)
        gsems = (g0, g1)
        wsems = (w0, w1)

        def start_gather(k, buf, gsem):
            pltpu.async_copy(
                table_sh.at[idx_v.at[pl.ds(k * _C, _C)]], buf, gsem)

        def start_write(k, buf, wsem):
            obase = pl.multiple_of(base + k * _C, 8)
            pltpu.async_copy(buf, o_hbm.at[pl.ds(obase, _C)], wsem)

        start_gather(0, bufs[0], gsems[0])
        for k in range(_NMAIN):
            j, jn = k % 2, (k + 1) % 2
            pltpu.make_async_copy(table_sh.at[idx_v.at[pl.ds(0, _C)]],
                                  bufs[j], gsems[j]).wait()
            start_write(k, bufs[j], wsems[j])
            if k + 1 < _NMAIN:
                if k + 1 >= 2:
                    pltpu.make_async_copy(bufs[jn],
                                          o_hbm.at[pl.ds(0, _C)],
                                          wsems[jn]).wait()
                start_gather(k + 1, bufs[jn], gsems[jn])

        jt = _NMAIN % 2
        pltpu.make_async_copy(bufs[jt], o_hbm.at[pl.ds(0, _C)],
                              wsems[jt]).wait()
        tbase = pl.multiple_of(base + _NMAIN * _C, 8)

        @pl.when(w < 20)
        def _():
            pltpu.async_copy(
                table_sh.at[idx_v.at[pl.ds(_NMAIN * _C, _BIG - _NMAIN * _C)]],
                bufs[jt].at[pl.ds(0, _BIG - _NMAIN * _C)], gsems[jt])
            pltpu.make_async_copy(
                table_sh.at[idx_v.at[pl.ds(0, _BIG - _NMAIN * _C)]],
                bufs[jt].at[pl.ds(0, _BIG - _NMAIN * _C)], gsems[jt]).wait()
            pltpu.async_copy(bufs[jt].at[pl.ds(0, _BIG - _NMAIN * _C)],
                             o_hbm.at[pl.ds(tbase, _BIG - _NMAIN * _C)],
                             wsems[jt])

        @pl.when(w >= 20)
        def _():
            pltpu.async_copy(
                table_sh.at[idx_v.at[pl.ds(_NMAIN * _C, _SMALL - _NMAIN * _C)]],
                bufs[jt].at[pl.ds(0, _SMALL - _NMAIN * _C)], gsems[jt])
            pltpu.make_async_copy(
                table_sh.at[idx_v.at[pl.ds(0, _SMALL - _NMAIN * _C)]],
                bufs[jt].at[pl.ds(0, _SMALL - _NMAIN * _C)], gsems[jt]).wait()
            pltpu.async_copy(bufs[jt].at[pl.ds(0, _SMALL - _NMAIN * _C)],
                             o_hbm.at[pl.ds(tbase, _SMALL - _NMAIN * _C)],
                             wsems[jt])

        pltpu.make_async_copy(bufs[(_NMAIN - 1) % 2], o_hbm.at[pl.ds(0, _C)],
                              wsems[(_NMAIN - 1) % 2]).wait()

        @pl.when(w < 20)
        def _():
            pltpu.make_async_copy(bufs[jt].at[pl.ds(0, _BIG - _NMAIN * _C)],
                                  o_hbm.at[pl.ds(0, _BIG - _NMAIN * _C)],
                                  wsems[jt]).wait()

        @pl.when(w >= 20)
        def _():
            pltpu.make_async_copy(bufs[jt].at[pl.ds(0, _SMALL - _NMAIN * _C)],
                                  o_hbm.at[pl.ds(0, _SMALL - _NMAIN * _C)],
                                  wsems[jt]).wait()

    return _gather(table, idx)
